# baseline (speedup 1.0000x reference)
import functools

import numpy as np

import jax
import jax.numpy as jnp
from jax import lax
from jax.experimental import pallas as pl
from jax.experimental.pallas import tpu as pltpu
from jax.experimental.pallas import tpu_sc as plsc

N = 50000
E = 800000
H = 128
HE = 32

NW = 32
WIN = 1024
SUB = 256
NWIN = 25
EPT = WIN * NWIN
EPAD = NW * EPT
EROWS = EPAD // 128
RPT = EROWS // NW
EP4 = EPAD // 4

CH = 7296
NCHUNK = 7
DUMPB = CH
ACCR = CH + 128
ZPT = ACCR // 16
WPT = CH // 16
W3A = 392
W3L = (N - 6 * CH) - 15 * W3A

_HIGH = jax.lax.Precision.HIGHEST

NBLK = 256
NGRID = (N + NBLK - 1) // NBLK
EB2 = 64
EG2 = EPAD // (EB2 * 32)
PBLK = 512
EG4 = (E + 4 * PBLK - 1) // (4 * PBLK)


@functools.cache
def _mesh():
    return plsc.VectorSubcoreMesh(core_axis_name="c", subcore_axis_name="s")


def _ln(h, g, b):
    mu = jnp.mean(h, axis=-1, keepdims=True)
    d = h - mu
    var = jnp.mean(d * d, axis=-1, keepdims=True)
    return d * jax.lax.rsqrt(var + 1e-5) * g + b


def _gln(h, M, g4, b4):
    mu = _dot(h, M)
    d = h - mu
    var = _dot(d * d, M)
    return d * jax.lax.rsqrt(var + 1e-5) * g4 + b4


def _dot(a, b):
    return jnp.dot(a, b, precision=_HIGH)


def _full(shape):
    nd = len(shape)
    return pl.BlockSpec(shape, lambda i: (0,) * nd)


def _f32(shape):
    return jax.ShapeDtypeStruct(shape, jnp.float32)


def _nspec(w=H):
    return pl.BlockSpec((NBLK, w), lambda i: (i, 0))


def _t1_body(x, ne_W, ne_b, ne_g, ne_bb, tc_Wmsg, nf_o, a_o):
    h = jnp.maximum(_dot(x[...], ne_W[...]) + ne_b[...], 0.0)
    nf = _ln(h, ne_g[...], ne_bb[...])
    nf_o[...] = nf
    a_o[...] = _dot(nf, tc_Wmsg[...])


def _t2_body(ea, Wbig, b4, M, g4, bb4, Em, bm, ef_o, em_o):
    a = ea[...]
    y = jnp.stack([_dot(a, Wbig[k]) for k in range(8)], axis=1)
    y = y.reshape(8 * EB2, H)
    ef = _gln(jnp.maximum(y + b4[...], 0.0), M[...], g4[...], bb4[...])
    ef_o[...] = ef
    em = jnp.stack([_dot(ef, Em[j]) for j in range(4)], axis=1)
    em_o[...] = em.reshape(4 * 8 * EB2, H) + bm[...]


def _node_post(nf_r, agg_r, W_upd, W_self, b_upd, ln_g, ln_b, w_sc):
    nf = nf_r[...]
    h = jnp.maximum(_dot(agg_r[...], W_upd[...]) + _dot(nf, W_self[...])
                    + b_upd[...], 0.0)
    xn = _ln(nf + h, ln_g[...], ln_b[...])
    return xn, jax.nn.sigmoid(_dot(xn, w_sc[...]))


def _t3_body(nf, agg, W_upd, W_self, b_upd, ln_g, ln_b, w_sc, an_Wmsg, W_ee,
             nf_o, sc_o, a_o, p_o):
    xn, score = _node_post(nf, agg, W_upd, W_self, b_upd, ln_g, ln_b, w_sc)
    nf_o[...] = xn
    sc_o[...] = score
    a_o[...] = _dot(xn, an_Wmsg[...])
    p_o[...] = _dot(xn, W_ee[...])


def _t4_body(ef, S, BDeue, be4, M, g4, bb4, Em, bm, U, V, eop_b,
             ef_o, em_o, eops_o, efn_o):
    ef0 = ef[...]
    eu = jnp.maximum(S[...] + _dot(ef0, BDeue[...]) + be4[...], 0.0)
    ef1 = _gln(ef0 + eu, M[...], g4[...], bb4[...])
    ef_o[...] = ef1
    em = jnp.stack([_dot(ef1, Em[j]) for j in range(4)], axis=1)
    em_o[...] = em.reshape(4 * PBLK, H) + bm[...]
    eo = jnp.stack([_dot(ef1, V[j]) for j in range(4)], axis=1)
    eops_o[...] = eo.reshape(4 * PBLK, 3) + eop_b[...]
    en = jnp.stack([_dot(ef1, U[j]) for j in range(4)], axis=1)
    efn_o[...] = en.reshape(4 * PBLK, HE)


def _t5_body(nf, agg, W_upd, W_self, b_upd, ln_g, ln_b, w_sc, ce_Wmsg,
             nW1, nb1, nW2, nb2, cW1, cb1, cW2, cb2,
             nf_o, sc_o, a_o, nop_o, ncr_o):
    xn, score = _node_post(nf, agg, W_upd, W_self, b_upd, ln_g, ln_b, w_sc)
    nf_o[...] = xn
    sc_o[...] = score
    a_o[...] = _dot(xn, ce_Wmsg[...])
    nop_o[...] = _dot(jnp.maximum(_dot(xn, nW1[...]) + nb1[...], 0.0), nW2[...]) + nb2[...]
    ncr_o[...] = _dot(jnp.maximum(_dot(xn, cW1[...]) + cb1[...], 0.0), cW2[...]) + cb2[...]


def _t6_body(nf, agg, W_upd, W_self, b_upd, ln_g, ln_b, w_sc, sc_o):
    _, score = _node_post(nf, agg, W_upd, W_self, b_upd, ln_g, ln_b, w_sc)
    sc_o[...] = score


def _segsum_kernel(has_emb):
    scratch = [
        pltpu.VMEM((8, 128), jnp.int32),
        pltpu.VMEM((8, 128), jnp.int32),
        pltpu.VMEM((8, 128), jnp.int32),
        pltpu.VMEM((SUB, H), jnp.float32),
        pltpu.VMEM((SUB, H), jnp.float32),
        pltpu.VMEM_SHARED((ACCR, H), jnp.float32),
        pltpu.VMEM((16, H), jnp.float32),
        pltpu.VMEM((H,), jnp.float32),
        pltpu.SemaphoreType.DMA,
    ]

    def body(a_hbm, emb_hbm, src_hbm, dst_hbm, out_hbm,
             srcw, dstw, dlw, bufE, bufA, acc, zbuf, biasv, sem0):
        c = lax.axis_index("c")
        s = lax.axis_index("s")
        base_row = s * (EROWS // 16)
        base_edge = s * (EPAD // 16)
        lane = lax.iota(jnp.int32, 16)

        for i in range(16):
            for g in range(8):
                zbuf[i, pl.ds(g * 16, 16)] = jnp.zeros((16,), jnp.float32)
        if not has_emb:
            pltpu.sync_copy(emb_hbm, biasv)

        def round_body(r):
            ch = c * 4 + r
            lo = ch * CH

            def zero_body(z, _):
                pltpu.sync_copy(zbuf, acc.at[pl.ds(s * ZPT + z * 16, 16)])
                return _
            lax.fori_loop(0, ZPT // 16, zero_body, 0)
            plsc.subcore_barrier()

            def win_body(w, _):
                wb = base_row + 8 * w
                gb = base_edge + w * WIN
                pltpu.sync_copy(src_hbm.at[pl.ds(wb, 8)], srcw)
                pltpu.sync_copy(dst_hbm.at[pl.ds(wb, 8)], dstw)
                for k in range(8):
                    for v in range(8):
                        sl = pl.ds(v * 16, 16)
                        dv = dstw[k, sl]
                        inb = (dv >= lo) & (dv < lo + CH)
                        dump = DUMPB + v * 16 + lane
                        dlw[k, sl] = jnp.where(inb, dv - lo, dump)
                for sw in range(4):
                    if has_emb:
                        pltpu.sync_copy(
                            emb_hbm.at[pl.ds(gb + sw * SUB, SUB)], bufE)
                    for k2 in range(2):
                        pltpu.async_copy(
                            a_hbm.at[srcw.at[2 * sw + k2]],
                            bufA.at[pl.ds(k2 * 128, 128)], sem0).wait()

                    def row_body(rr, _):
                        for g in range(8):
                            sl = pl.ds(g * 16, 16)
                            if has_emb:
                                m = bufE[rr, sl] + bufA[rr, sl]
                            else:
                                m = bufA[rr, sl] + biasv[sl]
                            bufE[rr, sl] = jnp.maximum(m, 0.0)
                        return _
                    lax.fori_loop(0, SUB, row_body, 0)

                    for k2 in range(2):
                        pltpu.sync_copy(bufE.at[pl.ds(k2 * 128, 128)],
                                        acc.at[dlw.at[2 * sw + k2]], add=True)
                return _
            lax.fori_loop(0, 2 * NWIN, win_body, 0)
            plsc.subcore_barrier()

            if r < 2:
                pltpu.sync_copy(acc.at[pl.ds(s * WPT, WPT)],
                                out_hbm.at[pl.ds(lo + s * WPT, WPT)])
            elif r == 2:
                @pl.when(c == 0)
                def _wb_full():
                    pltpu.sync_copy(acc.at[pl.ds(s * WPT, WPT)],
                                    out_hbm.at[pl.ds(lo + s * WPT, WPT)])

                @pl.when(c == 1)
                def _wb_tail():
                    @pl.when(s < 15)
                    def _wa():
                        pltpu.sync_copy(
                            acc.at[pl.ds(s * W3A, W3A)],
                            out_hbm.at[pl.ds(lo + s * W3A, W3A)])

                    @pl.when(s == 15)
                    def _wl():
                        pltpu.sync_copy(
                            acc.at[pl.ds(15 * W3A, W3L)],
                            out_hbm.at[pl.ds(lo + 15 * W3A, W3L)])
            else:
                pltpu.sync_copy(acc.at[pl.ds(s * WPT, WPT)],
                                out_hbm.at[pl.ds(lo + s * WPT, WPT)])
            plsc.subcore_barrier()

        for r in range(4):
            if r < 3:
                round_body(r)
            else:
                @pl.when(c == 0)
                def _last_round():
                    round_body(3)

    return pl.kernel(body, mesh=_mesh(), out_type=_f32((N, H)),
                     scratch_types=scratch)


_segsum_kernel = functools.cache(_segsum_kernel)


def _edge_gather_sum():
    scratch = [
        pltpu.VMEM((8, 128), jnp.int32),
        pltpu.VMEM((8, 128), jnp.int32),
        pltpu.VMEM((SUB, H), jnp.float32),
        pltpu.VMEM((SUB, H), jnp.float32),
        pltpu.VMEM((SUB // 4, H), jnp.float32),
        pltpu.SemaphoreType.DMA,
        pltpu.SemaphoreType.DMA,
    ]

    def body(p_hbm, src_hbm, dst_hbm, out_hbm, srcw, dstw, bufA, bufB, bufS,
             sem0, sem1):
        c = lax.axis_index("c")
        s = lax.axis_index("s")
        wid = s * 2 + c
        base_row = wid * RPT
        base_p4 = wid * (EPT // 4)

        def win_body(w, _):
            wb = base_row + 8 * w
            pltpu.sync_copy(src_hbm.at[pl.ds(wb, 8)], srcw)
            pltpu.sync_copy(dst_hbm.at[pl.ds(wb, 8)], dstw)
            for sw in range(4):
                for k2 in range(2):
                    pltpu.async_copy(p_hbm.at[srcw.at[2 * sw + k2]],
                                     bufA.at[pl.ds(k2 * 128, 128)], sem0).wait()
                    pltpu.async_copy(p_hbm.at[dstw.at[2 * sw + k2]],
                                     bufB.at[pl.ds(k2 * 128, 128)], sem1).wait()

                def row_body(rr, _):
                    for j in range(4):
                        for g in range(2):
                            a = bufA[4 * rr + j, pl.ds(g * 16, 16)]
                            b = bufB[4 * rr + j, pl.ds(32 + g * 16, 16)]
                            bufS[rr, pl.ds(32 * j + g * 16, 16)] = a + b
                    return _
                lax.fori_loop(0, SUB // 4, row_body, 0)
                pltpu.sync_copy(
                    bufS, out_hbm.at[pl.ds(base_p4 + (w * 4 + sw) * (SUB // 4),
                                           SUB // 4)])
            return _
        lax.fori_loop(0, NWIN, win_body, 0)

    return pl.kernel(body, mesh=_mesh(), out_type=_f32((EP4, H)),
                     scratch_types=scratch)


_edge_gather_sum = functools.cache(_edge_gather_sum)


def _tile4(v):
    return jnp.tile(v, 4)


def _packed_weights(p):
    f32 = jnp.float32
    z = jnp.zeros
    eeW = p['ee_W']
    B = jax.scipy.linalg.block_diag(eeW, eeW, eeW, eeW)
    Wbig = jnp.stack([z((H, H), f32).at[16 * k:16 * k + 16, :].set(B)
                      for k in range(8)])
    M = jax.scipy.linalg.block_diag(
        *([jnp.ones((HE, HE), f32) / HE] * 4))
    BDeue = jax.scipy.linalg.block_diag(*([p['tc_W_eue']] * 4))
    Em_tc = jnp.stack([z((H, H), f32).at[32 * j:32 * j + 32, :]
                       .set(p['tc_W_emsg']) for j in range(4)])
    Em_an = jnp.stack([z((H, H), f32).at[32 * j:32 * j + 32, :]
                       .set(p['an_W_emsg']) for j in range(4)])
    eye = jnp.eye(HE, dtype=f32)
    U = jnp.stack([z((H, HE), f32).at[32 * j:32 * j + 32, :].set(eye)
                   for j in range(4)])
    V = jnp.stack([z((H, 3), f32).at[32 * j:32 * j + 32, :].set(p['eop_W'])
                   for j in range(4)])
    Wee = jnp.concatenate(
        [p['tc_W_eus'], p['tc_W_eud'], z((H, 64), f32)], axis=1)
    return dict(Wbig=Wbig, M=M, BDeue=BDeue, Em_tc=Em_tc, Em_an=Em_an,
                U=U, V=V, Wee=Wee)


def kernel(x, edge_index, edge_attr, params):
    p = params
    w = _packed_weights(p)
    src = edge_index[0]
    dst = edge_index[1]
    padn = EPAD - E
    src2d = jnp.concatenate(
        [src, jnp.zeros((padn,), jnp.int32)]).reshape(EROWS, 128)
    dst2d = jnp.concatenate(
        [dst, jnp.full((padn,), N, jnp.int32)]).reshape(EROWS, 128)
    ea2 = jnp.concatenate(
        [edge_attr, jnp.zeros((padn, 4), jnp.float32)]).reshape(EPAD // 32, 128)

    nf0, A1 = pl.pallas_call(
        _t1_body, grid=(NGRID,),
        in_specs=[pl.BlockSpec((NBLK, 7), lambda i: (i, 0)), _full((7, H)),
                  _full((H,)), _full((H,)), _full((H,)), _full((H, H))],
        out_specs=[_nspec(), _nspec()],
        out_shape=[_f32((N, H)), _f32((N, H))],
    )(x, p['ne_W'], p['ne_b'], p['ne_ln_g'], p['ne_ln_b'], p['tc_W_msg'])

    ef0p, EMb1 = pl.pallas_call(
        _t2_body, grid=(EG2,),
        in_specs=[pl.BlockSpec((EB2, 128), lambda i: (i, 0)),
                  _full((8, H, H)), _full((H,)), _full((H, H)), _full((H,)),
                  _full((H,)), _full((4, H, H)), _full((H,))],
        out_specs=[pl.BlockSpec((8 * EB2, H), lambda i: (i, 0)),
                   pl.BlockSpec((32 * EB2, H), lambda i: (i, 0))],
        out_shape=[_f32((EP4, H)), _f32((EPAD, H))],
    )(ea2, w['Wbig'], _tile4(p['ee_b']), w['M'], _tile4(p['ee_ln_g']),
      _tile4(p['ee_ln_b']), w['Em_tc'], p['tc_b_msg'])

    agg1 = _segsum_kernel(True)(A1, EMb1, src2d, dst2d)

    nf1, topo, A2, P = pl.pallas_call(
        _t3_body, grid=(NGRID,),
        in_specs=[_nspec(), _nspec(), _full((H, H)), _full((H, H)),
                  _full((H,)), _full((H,)), _full((H,)), _full((H, 1)),
                  _full((H, H)), _full((H, H))],
        out_specs=[_nspec(), _nspec(1), _nspec(), _nspec()],
        out_shape=[_f32((N, H)), _f32((N, 1)), _f32((N, H)), _f32((N, H))],
    )(nf0, agg1, p['tc_W_upd'], p['tc_W_self'], p['tc_b_upd'],
      p['tc_ln_g'], p['tc_ln_b'], p['tc_w_score'], p['an_W_msg'], w['Wee'])

    Sp = _edge_gather_sum()(P, src2d, dst2d)

    ef1p, EMb2, eops, ef1 = pl.pallas_call(
        _t4_body, grid=(EG4,),
        in_specs=[pl.BlockSpec((PBLK, H), lambda i: (i, 0)),
                  pl.BlockSpec((PBLK, H), lambda i: (i, 0)),
                  _full((H, H)), _full((H,)), _full((H, H)), _full((H,)),
                  _full((H,)), _full((4, H, H)), _full((H,)),
                  _full((4, H, HE)), _full((4, H, 3)), _full((3,))],
        out_specs=[pl.BlockSpec((PBLK, H), lambda i: (i, 0)),
                   pl.BlockSpec((4 * PBLK, H), lambda i: (i, 0)),
                   pl.BlockSpec((4 * PBLK, 3), lambda i: (i, 0)),
                   pl.BlockSpec((4 * PBLK, HE), lambda i: (i, 0))],
        out_shape=[_f32((EP4, H)), _f32((EPAD, H)), _f32((E, 3)),
                   _f32((E, HE))],
    )(ef0p, Sp, w['BDeue'], _tile4(p['tc_b_eupd']), w['M'],
      _tile4(p['tc_eln_g']), _tile4(p['tc_eln_b']), w['Em_an'],
      p['an_b_msg'], w['U'], w['V'], p['eop_b'])

    agg2 = _segsum_kernel(True)(A2, EMb2, src2d, dst2d)

    nf2, anat, A3, nops, ncorr = pl.pallas_call(
        _t5_body, grid=(NGRID,),
        in_specs=[_nspec(), _nspec(), _full((H, H)), _full((H, H)),
                  _full((H,)), _full((H,)), _full((H,)), _full((H, 1)),
                  _full((H, H)), _full((H, H // 2)), _full((H // 2,)),
                  _full((H // 2, 3)), _full((3,)), _full((H, H // 2)),
                  _full((H // 2,)), _full((H // 2, 4)), _full((4,))],
        out_specs=[_nspec(), _nspec(1), _nspec(), _nspec(3), _nspec(4)],
        out_shape=[_f32((N, H)), _f32((N, 1)), _f32((N, H)),
                   _f32((N, 3)), _f32((N, 4))],
    )(nf1, agg2, p['an_W_upd'], p['an_W_self'], p['an_b_upd'],
      p['an_ln_g'], p['an_ln_b'], p['an_w_score'], p['ce_W_msg'],
      p['nop_W1'], p['nop_b1'], p['nop_W2'], p['nop_b2'],
      p['ncr_W1'], p['ncr_b1'], p['ncr_W2'], p['ncr_b2'])

    agg3 = _segsum_kernel(False)(A3, p['ce_b_msg'], src2d, dst2d)

    cons = pl.pallas_call(
        _t6_body, grid=(NGRID,),
        in_specs=[_nspec(), _nspec(), _full((H, H)), _full((H, H)),
                  _full((H,)), _full((H,)), _full((H,)), _full((H, 1))],
        out_specs=_nspec(1),
        out_shape=_f32((N, 1)),
    )(nf2, agg3, p['ce_W_upd'], p['ce_W_self'], p['ce_b_upd'],
      p['ce_ln_g'], p['ce_ln_b'], p['ce_w_score'])

    return (nops, ncorr, eops, nf2, ef1, topo, anat, cons)


# speedup vs baseline: 1.1949x; 1.1949x over previous
import functools

import numpy as np

import jax
import jax.numpy as jnp
from jax import lax
from jax.experimental import pallas as pl
from jax.experimental.pallas import tpu as pltpu
from jax.experimental.pallas import tpu_sc as plsc

N = 50000
E = 800000
H = 128
HE = 32

NW = 32
WIN = 1024
SUB = 128
NWIN = 25
EPT = WIN * NWIN
EPAD = NW * EPT
EROWS = EPAD // 128
RPT = EROWS // NW
EP4 = EPAD // 4

CH = 7296
NCHUNK = 7
DUMPB = CH
ACCR = CH + 128
ZPT = ACCR // 16
WPT = CH // 16
W3A = 392
W3L = (N - 6 * CH) - 15 * W3A

_HIGH = jax.lax.Precision.HIGHEST

NBLK = 256
NGRID = (N + NBLK - 1) // NBLK
EB2 = 64
EG2 = EPAD // (EB2 * 32)
PBLK = 512
EG4 = (E + 4 * PBLK - 1) // (4 * PBLK)


@functools.cache
def _mesh():
    return plsc.VectorSubcoreMesh(core_axis_name="c", subcore_axis_name="s")


def _ln(h, g, b):
    mu = jnp.mean(h, axis=-1, keepdims=True)
    d = h - mu
    var = jnp.mean(d * d, axis=-1, keepdims=True)
    return d * jax.lax.rsqrt(var + 1e-5) * g + b


def _gln(h, M, g4, b4):
    mu = _dot(h, M)
    d = h - mu
    var = _dot(d * d, M)
    return d * jax.lax.rsqrt(var + 1e-5) * g4 + b4


def _dot(a, b):
    return jnp.dot(a, b, precision=_HIGH)


def _full(shape):
    nd = len(shape)
    return pl.BlockSpec(shape, lambda i: (0,) * nd)


def _f32(shape):
    return jax.ShapeDtypeStruct(shape, jnp.float32)


def _nspec(w=H):
    return pl.BlockSpec((NBLK, w), lambda i: (i, 0))


def _t1_body(x, ne_W, ne_b, ne_g, ne_bb, tc_Wmsg, nf_o, a_o):
    h = jnp.maximum(_dot(x[...], ne_W[...]) + ne_b[...], 0.0)
    nf = _ln(h, ne_g[...], ne_bb[...])
    nf_o[...] = nf
    a_o[...] = _dot(nf, tc_Wmsg[...])


def _t2_body(ea, Wbig, b4, M, g4, bb4, Em, bm, ef_o, em_o):
    a = ea[...]
    y = jnp.stack([_dot(a, Wbig[k]) for k in range(8)], axis=1)
    y = y.reshape(8 * EB2, H)
    ef = _gln(jnp.maximum(y + b4[...], 0.0), M[...], g4[...], bb4[...])
    ef_o[...] = ef
    em = jnp.stack([_dot(ef, Em[j]) for j in range(4)], axis=1)
    em_o[...] = em.reshape(4 * 8 * EB2, H) + bm[...]


def _node_post(nf_r, agg_r, W_upd, W_self, b_upd, ln_g, ln_b, w_sc):
    nf = nf_r[...]
    h = jnp.maximum(_dot(agg_r[...], W_upd[...]) + _dot(nf, W_self[...])
                    + b_upd[...], 0.0)
    xn = _ln(nf + h, ln_g[...], ln_b[...])
    return xn, jax.nn.sigmoid(_dot(xn, w_sc[...]))


def _t3_body(nf, agg, W_upd, W_self, b_upd, ln_g, ln_b, w_sc, an_Wmsg, W_ee,
             nf_o, sc_o, a_o, p_o):
    xn, score = _node_post(nf, agg, W_upd, W_self, b_upd, ln_g, ln_b, w_sc)
    nf_o[...] = xn
    sc_o[...] = score
    a_o[...] = _dot(xn, an_Wmsg[...])
    p_o[...] = _dot(xn, W_ee[...])


def _t4_body(ef, S, BDeue, be4, M, g4, bb4, Em, bm, U, V, eop_b,
             ef_o, em_o, eops_o, efn_o):
    ef0 = ef[...]
    eu = jnp.maximum(S[...] + _dot(ef0, BDeue[...]) + be4[...], 0.0)
    ef1 = _gln(ef0 + eu, M[...], g4[...], bb4[...])
    ef_o[...] = ef1
    em = jnp.stack([_dot(ef1, Em[j]) for j in range(4)], axis=1)
    em_o[...] = em.reshape(4 * PBLK, H) + bm[...]
    eo = jnp.stack([_dot(ef1, V[j]) for j in range(4)], axis=1)
    eops_o[...] = eo.reshape(4 * PBLK, 3) + eop_b[...]
    en = jnp.stack([_dot(ef1, U[j]) for j in range(4)], axis=1)
    efn_o[...] = en.reshape(4 * PBLK, HE)


def _t5_body(nf, agg, W_upd, W_self, b_upd, ln_g, ln_b, w_sc, ce_Wmsg,
             nW1, nb1, nW2, nb2, cW1, cb1, cW2, cb2,
             nf_o, sc_o, a_o, nop_o, ncr_o):
    xn, score = _node_post(nf, agg, W_upd, W_self, b_upd, ln_g, ln_b, w_sc)
    nf_o[...] = xn
    sc_o[...] = score
    a_o[...] = _dot(xn, ce_Wmsg[...])
    nop_o[...] = _dot(jnp.maximum(_dot(xn, nW1[...]) + nb1[...], 0.0), nW2[...]) + nb2[...]
    ncr_o[...] = _dot(jnp.maximum(_dot(xn, cW1[...]) + cb1[...], 0.0), cW2[...]) + cb2[...]


def _t6_body(nf, agg, W_upd, W_self, b_upd, ln_g, ln_b, w_sc, sc_o):
    _, score = _node_post(nf, agg, W_upd, W_self, b_upd, ln_g, ln_b, w_sc)
    sc_o[...] = score


def _segsum_kernel(has_emb):
    scratch = [
        pltpu.VMEM((8, 128), jnp.int32),
        pltpu.VMEM((8, 128), jnp.int32),
        pltpu.VMEM((8, 128), jnp.int32),
        pltpu.VMEM((SUB, H), jnp.float32),
        pltpu.VMEM((SUB, H), jnp.float32),
        pltpu.VMEM((SUB, H), jnp.float32),
        pltpu.VMEM((SUB, H), jnp.float32),
        pltpu.VMEM_SHARED((ACCR, H), jnp.float32),
        pltpu.VMEM((16, H), jnp.float32),
        pltpu.VMEM((H,), jnp.float32),
        pltpu.SemaphoreType.DMA,
        pltpu.SemaphoreType.DMA,
    ]

    def body(a_hbm, emb_hbm, src_hbm, dst_hbm, out_hbm,
             srcw, dstw, dlw, bufE0, bufE1, bufA0, bufA1,
             acc, zbuf, biasv, semE, semA):
        c = lax.axis_index("c")
        s = lax.axis_index("s")
        base_row = s * (EROWS // 16)
        base_edge = s * (EPAD // 16)
        lane = lax.iota(jnp.int32, 16)
        bufE = (bufE0, bufE1)
        bufA = (bufA0, bufA1)

        for i in range(16):
            for g in range(8):
                zbuf[i, pl.ds(g * 16, 16)] = jnp.zeros((16,), jnp.float32)
        if not has_emb:
            pltpu.sync_copy(emb_hbm, biasv)

        def round_body(r):
            ch = c * 4 + r
            lo = ch * CH

            def zero_body(z, _):
                pltpu.sync_copy(zbuf, acc.at[pl.ds(s * ZPT + z * 16, 16)])
                return _
            lax.fori_loop(0, ZPT // 16, zero_body, 0)
            plsc.subcore_barrier()

            def win_body(w, _):
                wb = base_row + 8 * w
                gb = base_edge + w * WIN
                pltpu.sync_copy(src_hbm.at[pl.ds(wb, 8)], srcw)
                pltpu.sync_copy(dst_hbm.at[pl.ds(wb, 8)], dstw)
                for k in range(8):
                    for v in range(8):
                        sl = pl.ds(v * 16, 16)
                        dv = dstw[k, sl]
                        inb = (dv >= lo) & (dv < lo + CH)
                        dump = DUMPB + v * 16 + lane
                        dlw[k, sl] = jnp.where(inb, dv - lo, dump)

                def issue(sub):
                    pb = sub % 2
                    dE = None
                    if has_emb:
                        dE = pltpu.async_copy(
                            emb_hbm.at[pl.ds(gb + sub * SUB, SUB)],
                            bufE[pb], semE)
                    dA = pltpu.async_copy(a_hbm.at[srcw.at[sub]], bufA[pb],
                                          semA)
                    return dE, dA

                descs = [None] * 8
                descs[0] = issue(0)
                for sub in range(8):
                    pb = sub % 2
                    if sub < 7:
                        descs[sub + 1] = issue(sub + 1)
                    dE, dA = descs[sub]
                    if dE is not None:
                        dE.wait()
                    dA.wait()

                    def row_body(rr, _):
                        for g in range(8):
                            sl = pl.ds(g * 16, 16)
                            if has_emb:
                                m = bufE[pb][rr, sl] + bufA[pb][rr, sl]
                            else:
                                m = bufA[pb][rr, sl] + biasv[sl]
                            bufA[pb][rr, sl] = jnp.maximum(m, 0.0)
                        return _
                    lax.fori_loop(0, SUB, row_body, 0)
                    pltpu.sync_copy(bufA[pb], acc.at[dlw.at[sub]], add=True)
                return _
            lax.fori_loop(0, 2 * NWIN, win_body, 0)
            plsc.subcore_barrier()

            if r < 2:
                pltpu.sync_copy(acc.at[pl.ds(s * WPT, WPT)],
                                out_hbm.at[pl.ds(lo + s * WPT, WPT)])
            elif r == 2:
                @pl.when(c == 0)
                def _wb_full():
                    pltpu.sync_copy(acc.at[pl.ds(s * WPT, WPT)],
                                    out_hbm.at[pl.ds(lo + s * WPT, WPT)])

                @pl.when(c == 1)
                def _wb_tail():
                    @pl.when(s < 15)
                    def _wa():
                        pltpu.sync_copy(
                            acc.at[pl.ds(s * W3A, W3A)],
                            out_hbm.at[pl.ds(lo + s * W3A, W3A)])

                    @pl.when(s == 15)
                    def _wl():
                        pltpu.sync_copy(
                            acc.at[pl.ds(15 * W3A, W3L)],
                            out_hbm.at[pl.ds(lo + 15 * W3A, W3L)])
            else:
                pltpu.sync_copy(acc.at[pl.ds(s * WPT, WPT)],
                                out_hbm.at[pl.ds(lo + s * WPT, WPT)])
            plsc.subcore_barrier()

        for r in range(4):
            if r < 3:
                round_body(r)
            else:
                @pl.when(c == 0)
                def _last_round():
                    round_body(3)

    return pl.kernel(body, mesh=_mesh(), out_type=_f32((N, H)),
                     scratch_types=scratch)


_segsum_kernel = functools.cache(_segsum_kernel)


def _edge_gather_sum():
    scratch = [
        pltpu.VMEM((8, 128), jnp.int32),
        pltpu.VMEM((8, 128), jnp.int32),
        pltpu.VMEM((256, H), jnp.float32),
        pltpu.VMEM((256, H), jnp.float32),
        pltpu.VMEM((64, H), jnp.float32),
        pltpu.SemaphoreType.DMA,
        pltpu.SemaphoreType.DMA,
    ]

    def body(p_hbm, src_hbm, dst_hbm, out_hbm, srcw, dstw, bufA, bufB, bufS,
             sem0, sem1):
        c = lax.axis_index("c")
        s = lax.axis_index("s")
        wid = s * 2 + c
        base_row = wid * RPT
        base_p4 = wid * (EPT // 4)

        def win_body(w, _):
            wb = base_row + 8 * w
            pltpu.sync_copy(src_hbm.at[pl.ds(wb, 8)], srcw)
            pltpu.sync_copy(dst_hbm.at[pl.ds(wb, 8)], dstw)
            for sw in range(4):
                for k2 in range(2):
                    pltpu.async_copy(p_hbm.at[srcw.at[2 * sw + k2]],
                                     bufA.at[pl.ds(k2 * 128, 128)], sem0).wait()
                    pltpu.async_copy(p_hbm.at[dstw.at[2 * sw + k2]],
                                     bufB.at[pl.ds(k2 * 128, 128)], sem1).wait()

                def row_body(rr, _):
                    for j in range(4):
                        for g in range(2):
                            a = bufA[4 * rr + j, pl.ds(g * 16, 16)]
                            b = bufB[4 * rr + j, pl.ds(32 + g * 16, 16)]
                            bufS[rr, pl.ds(32 * j + g * 16, 16)] = a + b
                    return _
                lax.fori_loop(0, 64, row_body, 0)
                pltpu.sync_copy(
                    bufS, out_hbm.at[pl.ds(base_p4 + (w * 4 + sw) * 64, 64)])
            return _
        lax.fori_loop(0, NWIN, win_body, 0)

    return pl.kernel(body, mesh=_mesh(), out_type=_f32((EP4, H)),
                     scratch_types=scratch)


_edge_gather_sum = functools.cache(_edge_gather_sum)


def _tile4(v):
    return jnp.tile(v, 4)


def _packed_weights(p):
    f32 = jnp.float32
    z = jnp.zeros
    eeW = p['ee_W']
    B = jax.scipy.linalg.block_diag(eeW, eeW, eeW, eeW)
    Wbig = jnp.stack([z((H, H), f32).at[16 * k:16 * k + 16, :].set(B)
                      for k in range(8)])
    M = jax.scipy.linalg.block_diag(
        *([jnp.ones((HE, HE), f32) / HE] * 4))
    BDeue = jax.scipy.linalg.block_diag(*([p['tc_W_eue']] * 4))
    Em_tc = jnp.stack([z((H, H), f32).at[32 * j:32 * j + 32, :]
                       .set(p['tc_W_emsg']) for j in range(4)])
    Em_an = jnp.stack([z((H, H), f32).at[32 * j:32 * j + 32, :]
                       .set(p['an_W_emsg']) for j in range(4)])
    eye = jnp.eye(HE, dtype=f32)
    U = jnp.stack([z((H, HE), f32).at[32 * j:32 * j + 32, :].set(eye)
                   for j in range(4)])
    V = jnp.stack([z((H, 3), f32).at[32 * j:32 * j + 32, :].set(p['eop_W'])
                   for j in range(4)])
    Wee = jnp.concatenate(
        [p['tc_W_eus'], p['tc_W_eud'], z((H, 64), f32)], axis=1)
    return dict(Wbig=Wbig, M=M, BDeue=BDeue, Em_tc=Em_tc, Em_an=Em_an,
                U=U, V=V, Wee=Wee)


def kernel(x, edge_index, edge_attr, params):
    p = params
    w = _packed_weights(p)
    src = edge_index[0]
    dst = edge_index[1]
    padn = EPAD - E
    src2d = jnp.concatenate(
        [src, jnp.zeros((padn,), jnp.int32)]).reshape(EROWS, 128)
    dst2d = jnp.concatenate(
        [dst, jnp.full((padn,), N, jnp.int32)]).reshape(EROWS, 128)
    ea2 = jnp.concatenate(
        [edge_attr, jnp.zeros((padn, 4), jnp.float32)]).reshape(EPAD // 32, 128)

    nf0, A1 = pl.pallas_call(
        _t1_body, grid=(NGRID,),
        in_specs=[pl.BlockSpec((NBLK, 7), lambda i: (i, 0)), _full((7, H)),
                  _full((H,)), _full((H,)), _full((H,)), _full((H, H))],
        out_specs=[_nspec(), _nspec()],
        out_shape=[_f32((N, H)), _f32((N, H))],
    )(x, p['ne_W'], p['ne_b'], p['ne_ln_g'], p['ne_ln_b'], p['tc_W_msg'])

    ef0p, EMb1 = pl.pallas_call(
        _t2_body, grid=(EG2,),
        in_specs=[pl.BlockSpec((EB2, 128), lambda i: (i, 0)),
                  _full((8, H, H)), _full((H,)), _full((H, H)), _full((H,)),
                  _full((H,)), _full((4, H, H)), _full((H,))],
        out_specs=[pl.BlockSpec((8 * EB2, H), lambda i: (i, 0)),
                   pl.BlockSpec((32 * EB2, H), lambda i: (i, 0))],
        out_shape=[_f32((EP4, H)), _f32((EPAD, H))],
    )(ea2, w['Wbig'], _tile4(p['ee_b']), w['M'], _tile4(p['ee_ln_g']),
      _tile4(p['ee_ln_b']), w['Em_tc'], p['tc_b_msg'])

    agg1 = _segsum_kernel(True)(A1, EMb1, src2d, dst2d)

    nf1, topo, A2, P = pl.pallas_call(
        _t3_body, grid=(NGRID,),
        in_specs=[_nspec(), _nspec(), _full((H, H)), _full((H, H)),
                  _full((H,)), _full((H,)), _full((H,)), _full((H, 1)),
                  _full((H, H)), _full((H, H))],
        out_specs=[_nspec(), _nspec(1), _nspec(), _nspec()],
        out_shape=[_f32((N, H)), _f32((N, 1)), _f32((N, H)), _f32((N, H))],
    )(nf0, agg1, p['tc_W_upd'], p['tc_W_self'], p['tc_b_upd'],
      p['tc_ln_g'], p['tc_ln_b'], p['tc_w_score'], p['an_W_msg'], w['Wee'])

    Sp = _edge_gather_sum()(P, src2d, dst2d)

    ef1p, EMb2, eops, ef1 = pl.pallas_call(
        _t4_body, grid=(EG4,),
        in_specs=[pl.BlockSpec((PBLK, H), lambda i: (i, 0)),
                  pl.BlockSpec((PBLK, H), lambda i: (i, 0)),
                  _full((H, H)), _full((H,)), _full((H, H)), _full((H,)),
                  _full((H,)), _full((4, H, H)), _full((H,)),
                  _full((4, H, HE)), _full((4, H, 3)), _full((3,))],
        out_specs=[pl.BlockSpec((PBLK, H), lambda i: (i, 0)),
                   pl.BlockSpec((4 * PBLK, H), lambda i: (i, 0)),
                   pl.BlockSpec((4 * PBLK, 3), lambda i: (i, 0)),
                   pl.BlockSpec((4 * PBLK, HE), lambda i: (i, 0))],
        out_shape=[_f32((EP4, H)), _f32((EPAD, H)), _f32((E, 3)),
                   _f32((E, HE))],
    )(ef0p, Sp, w['BDeue'], _tile4(p['tc_b_eupd']), w['M'],
      _tile4(p['tc_eln_g']), _tile4(p['tc_eln_b']), w['Em_an'],
      p['an_b_msg'], w['U'], w['V'], p['eop_b'])

    agg2 = _segsum_kernel(True)(A2, EMb2, src2d, dst2d)

    nf2, anat, A3, nops, ncorr = pl.pallas_call(
        _t5_body, grid=(NGRID,),
        in_specs=[_nspec(), _nspec(), _full((H, H)), _full((H, H)),
                  _full((H,)), _full((H,)), _full((H,)), _full((H, 1)),
                  _full((H, H)), _full((H, H // 2)), _full((H // 2,)),
                  _full((H // 2, 3)), _full((3,)), _full((H, H // 2)),
                  _full((H // 2,)), _full((H // 2, 4)), _full((4,))],
        out_specs=[_nspec(), _nspec(1), _nspec(), _nspec(3), _nspec(4)],
        out_shape=[_f32((N, H)), _f32((N, 1)), _f32((N, H)),
                   _f32((N, 3)), _f32((N, 4))],
    )(nf1, agg2, p['an_W_upd'], p['an_W_self'], p['an_b_upd'],
      p['an_ln_g'], p['an_ln_b'], p['an_w_score'], p['ce_W_msg'],
      p['nop_W1'], p['nop_b1'], p['nop_W2'], p['nop_b2'],
      p['ncr_W1'], p['ncr_b1'], p['ncr_W2'], p['ncr_b2'])

    agg3 = _segsum_kernel(False)(A3, p['ce_b_msg'], src2d, dst2d)

    cons = pl.pallas_call(
        _t6_body, grid=(NGRID,),
        in_specs=[_nspec(), _nspec(), _full((H, H)), _full((H, H)),
                  _full((H,)), _full((H,)), _full((H,)), _full((H, 1))],
        out_specs=_nspec(1),
        out_shape=_f32((N, 1)),
    )(nf2, agg3, p['ce_W_upd'], p['ce_W_self'], p['ce_b_upd'],
      p['ce_ln_g'], p['ce_ln_b'], p['ce_w_score'])

    return (nops, ncorr, eops, nf2, ef1, topo, anat, cons)
